# baseline (speedup 1.0000x reference)
import functools

import jax
import jax.numpy as jnp
from jax.experimental import pallas as pl
from jax.experimental.pallas import tpu as pltpu
from jax.experimental.pallas import tpu_sc as plsc

NUM_EMB = 16384
EMB_DIM = 512
FACT_DIM = 32
B = 8
PIX = 1024
NPIX = B * PIX
CB_TILE = 4096
NCB = NUM_EMB // CB_TILE

_PREC = jax.lax.Precision.DEFAULT

_SC_INFO = plsc.get_sparse_core_info()
_NW = _SC_INFO.num_cores * _SC_INFO.num_subcores
_BPW = NPIX // _NW


def _embnorm_kernel(embw_ref, embn_ref):
    ew = embw_ref[...]
    en = jnp.sqrt(jnp.sum(ew * ew, axis=1, keepdims=True))
    embn_ref[...] = ew / jnp.maximum(en, 1e-6)


def _search_kernel(x_ref, embn_ref, projw_ref, projb_ref,
                   idx_ref, loss_ref,
                   enc_scr, bval_scr, bidx_scr, loss_scr):
    b = pl.program_id(0)
    cb = pl.program_id(1)

    emb_n = embn_ref[...]

    @pl.when(cb == 0)
    def _proj():
        @pl.when(b == 0)
        def _init():
            loss_scr[...] = jnp.zeros_like(loss_scr)

        x = x_ref[0]
        enc = jax.lax.dot_general(projw_ref[...], x, (((1,), (0,)), ((), ())),
                                  precision=_PREC,
                                  preferred_element_type=jnp.float32)
        enc = enc + projb_ref[...]
        n = jnp.sqrt(jnp.sum(enc * enc, axis=0, keepdims=True))
        enc_scr[...] = enc / jnp.maximum(n, 1e-6)
        bval_scr[...] = jnp.full_like(bval_scr, -jnp.inf)
        bidx_scr[...] = jnp.zeros_like(bidx_scr)

    cos = jax.lax.dot_general(emb_n, enc_scr[...], (((1,), (0,)), ((), ())),
                              precision=_PREC,
                              preferred_element_type=jnp.float32)
    lmax = jnp.max(cos, axis=0, keepdims=True)
    lidx = jnp.argmax(cos, axis=0).reshape(1, PIX) + cb * CB_TILE
    better = lmax > bval_scr[...]
    bidx_scr[...] = jnp.where(better, lidx, bidx_scr[...])
    bval_scr[...] = jnp.where(better, lmax, bval_scr[...])

    @pl.when(cb == NCB - 1)
    def _finish_batch():
        idx_ref[0] = bidx_scr[0:1, :]
        loss_scr[...] += jnp.sum(2.0 - 2.0 * bval_scr[0:1, :]).reshape(1, 1)

        @pl.when(b == B - 1)
        def _finish_all():
            loss_ref[...] = loss_scr[...] / (B * FACT_DIM * PIX)


def _sc_body(idx_hbm, embw_hbm, lat_hbm, cnt_hbm,
             idx_v, rows_v, cnt_v, sem):
    wid = jax.lax.axis_index("s") * _SC_INFO.num_cores + jax.lax.axis_index("c")
    per_b = PIX // _BPW
    b = wid // per_b
    off = (wid % per_b) * _BPW
    pltpu.sync_copy(idx_hbm.at[b, 0, pl.ds(off, _BPW)], idx_v)
    pltpu.async_copy(embw_hbm.at[idx_v], rows_v, sem).wait()
    pltpu.sync_copy(rows_v, lat_hbm.at[pl.ds(wid * _BPW, _BPW)])
    zeros16 = jnp.zeros((16,), jnp.float32)

    def zbody(i, carry):
        cnt_v[pl.ds(i * 16, 16)] = zeros16
        return carry

    jax.lax.fori_loop(0, NUM_EMB // 16, zbody, 0)
    ones = jnp.ones((16,), jnp.float32)

    def body(i, carry):
        v = idx_v[pl.ds(i * 16, 16)]
        plsc.addupdate_scatter(cnt_v, [v], ones)
        return carry

    jax.lax.fori_loop(0, _BPW // 16, body, 0)
    pltpu.sync_copy(cnt_v, cnt_hbm.at[wid])


@functools.partial(
    pl.kernel,
    mesh=plsc.VectorSubcoreMesh(core_axis_name="c", subcore_axis_name="s"),
    out_type=[
        jax.ShapeDtypeStruct((NPIX, FACT_DIM), jnp.float32),
        jax.ShapeDtypeStruct((_NW, NUM_EMB), jnp.float32),
    ],
    scratch_types=[
        pltpu.VMEM((_BPW,), jnp.int32),
        pltpu.VMEM((_BPW, FACT_DIM), jnp.float32),
        pltpu.VMEM((NUM_EMB,), jnp.float32),
        pltpu.SemaphoreType.DMA,
    ],
    compiler_params=pltpu.CompilerParams(needs_layout_passes=False,
                                         use_tc_tiling_on_sc=False),
)
def _sc_gather_count(idx_hbm, embw_hbm, lat_hbm, cnt_hbm,
                     idx_v, rows_v, cnt_v, sem):
    _sc_body(idx_hbm, embw_hbm, lat_hbm, cnt_hbm,
             idx_v, rows_v, cnt_v, sem)


def _expand_kernel(lat_ref, expw_ref, expb_ref, cnt_ref,
                   out_ref, perp_ref):
    b = pl.program_id(0)
    lat_n = lat_ref[0]
    lat_t = lat_n.T
    o = jax.lax.dot_general(expw_ref[...], lat_t, (((1,), (0,)), ((), ())),
                            precision=_PREC,
                            preferred_element_type=jnp.float32)
    out_ref[0] = o + expb_ref[...]

    @pl.when(b == 0)
    def _entropy():
        tot = jnp.sum(cnt_ref[...], axis=0, keepdims=True)
        u = tot / float(NPIX)
        ent = jnp.sum(-u * jnp.log(u + 1e-6))
        perp_ref[...] = jnp.exp(ent).reshape(1, 1)


def kernel(encodings, emb_w, proj_w, proj_b, exp_w, exp_b):
    x = encodings.reshape(B, EMB_DIM, PIX)
    pb = proj_b.reshape(FACT_DIM, 1)
    eb = exp_b.reshape(EMB_DIM, 1)

    emb_n = pl.pallas_call(
        _embnorm_kernel,
        grid=(NCB,),
        in_specs=[pl.BlockSpec((CB_TILE, FACT_DIM), lambda cb: (cb, 0))],
        out_specs=pl.BlockSpec((CB_TILE, FACT_DIM), lambda cb: (cb, 0)),
        out_shape=jax.ShapeDtypeStruct((NUM_EMB, FACT_DIM), jnp.float32),
    )(emb_w)

    idx, loss = pl.pallas_call(
        _search_kernel,
        grid=(B, NCB),
        in_specs=[
            pl.BlockSpec((1, EMB_DIM, PIX), lambda b, cb: (b, 0, 0)),
            pl.BlockSpec((CB_TILE, FACT_DIM), lambda b, cb: (cb, 0)),
            pl.BlockSpec((FACT_DIM, EMB_DIM), lambda b, cb: (0, 0)),
            pl.BlockSpec((FACT_DIM, 1), lambda b, cb: (0, 0)),
        ],
        out_specs=[
            pl.BlockSpec((1, 1, PIX), lambda b, cb: (b, 0, 0)),
            pl.BlockSpec((1, 1), lambda b, cb: (0, 0)),
        ],
        out_shape=[
            jax.ShapeDtypeStruct((B, 1, PIX), jnp.int32),
            jax.ShapeDtypeStruct((1, 1), jnp.float32),
        ],
        scratch_shapes=[
            pltpu.VMEM((FACT_DIM, PIX), jnp.float32),
            pltpu.VMEM((8, PIX), jnp.float32),
            pltpu.VMEM((8, PIX), jnp.int32),
            pltpu.VMEM((1, 1), jnp.float32),
        ],
        compiler_params=pltpu.CompilerParams(
            dimension_semantics=("arbitrary", "arbitrary"),
        ),
    )(x, emb_n, proj_w, pb)

    lat_raw, counts = _sc_gather_count(idx, emb_n)

    out, perp = pl.pallas_call(
        _expand_kernel,
        grid=(B,),
        in_specs=[
            pl.BlockSpec((1, PIX, FACT_DIM), lambda b: (b, 0, 0)),
            pl.BlockSpec((EMB_DIM, FACT_DIM), lambda b: (0, 0)),
            pl.BlockSpec((EMB_DIM, 1), lambda b: (0, 0)),
            pl.BlockSpec((_NW, NUM_EMB), lambda b: (0, 0)),
        ],
        out_specs=[
            pl.BlockSpec((1, EMB_DIM, PIX), lambda b: (b, 0, 0)),
            pl.BlockSpec((1, 1), lambda b: (0, 0)),
        ],
        out_shape=[
            jax.ShapeDtypeStruct((B, EMB_DIM, PIX), jnp.float32),
            jax.ShapeDtypeStruct((1, 1), jnp.float32),
        ],
        compiler_params=pltpu.CompilerParams(
            dimension_semantics=("arbitrary",),
        ),
    )(lat_raw.reshape(B, PIX, FACT_DIM), exp_w, eb, counts)

    return (out.reshape(B, EMB_DIM, 32, 32), loss[0, 0], perp[0, 0])


# speedup vs baseline: 1.0290x; 1.0290x over previous
import functools

import jax
import jax.numpy as jnp
from jax.experimental import pallas as pl
from jax.experimental.pallas import tpu as pltpu
from jax.experimental.pallas import tpu_sc as plsc

NUM_EMB = 16384
EMB_DIM = 512
FACT_DIM = 32
B = 8
PIX = 1024
NPIX = B * PIX
CB_TILE = 2048
NCB = NUM_EMB // CB_TILE

_PREC = jax.lax.Precision.DEFAULT

_SC_INFO = plsc.get_sparse_core_info()
_NW = _SC_INFO.num_cores * _SC_INFO.num_subcores
_BPW = NPIX // _NW


def _embnorm_kernel(embw_ref, embn_ref):
    ew = embw_ref[...]
    en = jnp.sqrt(jnp.sum(ew * ew, axis=1, keepdims=True))
    embn_ref[...] = ew / jnp.maximum(en, 1e-6)


def _search_kernel(x_ref, embn_ref, projw_ref, projb_ref,
                   idx_ref, loss_ref, loss_scr):
    b = pl.program_id(0)

    @pl.when(b == 0)
    def _init():
        loss_scr[...] = jnp.zeros_like(loss_scr)

    x = x_ref[0]
    enc = jax.lax.dot_general(projw_ref[...], x, (((1,), (0,)), ((), ())),
                              precision=_PREC,
                              preferred_element_type=jnp.float32)
    enc = enc + projb_ref[...]
    n = jnp.sqrt(jnp.sum(enc * enc, axis=0, keepdims=True))
    enc_n = enc / jnp.maximum(n, 1e-6)

    bval = jnp.full((1, PIX), -jnp.inf, jnp.float32)
    bidx = jnp.zeros((1, PIX), jnp.int32)
    for j in range(NCB):
        emb_n = embn_ref[j * CB_TILE:(j + 1) * CB_TILE, :]
        cos = jax.lax.dot_general(emb_n, enc_n, (((1,), (0,)), ((), ())),
                                  precision=_PREC,
                                  preferred_element_type=jnp.float32)
        lmax = jnp.max(cos, axis=0, keepdims=True)
        lidx = jnp.argmax(cos, axis=0).reshape(1, PIX) + j * CB_TILE
        better = lmax > bval
        bidx = jnp.where(better, lidx, bidx)
        bval = jnp.where(better, lmax, bval)

    idx_ref[0] = bidx
    loss_scr[...] += jnp.sum(2.0 - 2.0 * bval).reshape(1, 1)

    @pl.when(b == B - 1)
    def _finish_all():
        loss_ref[...] = loss_scr[...] / (B * FACT_DIM * PIX)


def _sc_body(idx_hbm, embw_hbm, lat_hbm, cnt_hbm,
             idx_v, rows_v, cnt_v, sem):
    wid = jax.lax.axis_index("s") * _SC_INFO.num_cores + jax.lax.axis_index("c")
    per_b = PIX // _BPW
    b = wid // per_b
    off = (wid % per_b) * _BPW
    pltpu.sync_copy(idx_hbm.at[b, 0, pl.ds(off, _BPW)], idx_v)
    pltpu.async_copy(embw_hbm.at[idx_v], rows_v, sem).wait()
    pltpu.sync_copy(rows_v, lat_hbm.at[pl.ds(wid * _BPW, _BPW)])
    zeros16 = jnp.zeros((16,), jnp.float32)

    def zbody(i, carry):
        cnt_v[pl.ds(i * 16, 16)] = zeros16
        return carry

    jax.lax.fori_loop(0, NUM_EMB // 16, zbody, 0)
    ones = jnp.ones((16,), jnp.float32)

    def body(i, carry):
        v = idx_v[pl.ds(i * 16, 16)]
        plsc.addupdate_scatter(cnt_v, [v], ones)
        return carry

    jax.lax.fori_loop(0, _BPW // 16, body, 0)
    pltpu.sync_copy(cnt_v, cnt_hbm.at[wid])


@functools.partial(
    pl.kernel,
    mesh=plsc.VectorSubcoreMesh(core_axis_name="c", subcore_axis_name="s"),
    out_type=[
        jax.ShapeDtypeStruct((NPIX, FACT_DIM), jnp.float32),
        jax.ShapeDtypeStruct((_NW, NUM_EMB), jnp.float32),
    ],
    scratch_types=[
        pltpu.VMEM((_BPW,), jnp.int32),
        pltpu.VMEM((_BPW, FACT_DIM), jnp.float32),
        pltpu.VMEM((NUM_EMB,), jnp.float32),
        pltpu.SemaphoreType.DMA,
    ],
    compiler_params=pltpu.CompilerParams(needs_layout_passes=False,
                                         use_tc_tiling_on_sc=False),
)
def _sc_gather_count(idx_hbm, embw_hbm, lat_hbm, cnt_hbm,
                     idx_v, rows_v, cnt_v, sem):
    _sc_body(idx_hbm, embw_hbm, lat_hbm, cnt_hbm,
             idx_v, rows_v, cnt_v, sem)


def _expand_kernel(lat_ref, expw_ref, expb_ref, cnt_ref,
                   out_ref, perp_ref):
    b = pl.program_id(0)
    lat_n = lat_ref[0]
    lat_t = lat_n.T
    o = jax.lax.dot_general(expw_ref[...], lat_t, (((1,), (0,)), ((), ())),
                            precision=_PREC,
                            preferred_element_type=jnp.float32)
    out_ref[0] = o + expb_ref[...]

    @pl.when(b == 0)
    def _entropy():
        tot = jnp.sum(cnt_ref[...], axis=0, keepdims=True)
        u = tot / float(NPIX)
        ent = jnp.sum(-u * jnp.log(u + 1e-6))
        perp_ref[...] = jnp.exp(ent).reshape(1, 1)


def kernel(encodings, emb_w, proj_w, proj_b, exp_w, exp_b):
    x = encodings.reshape(B, EMB_DIM, PIX)
    pb = proj_b.reshape(FACT_DIM, 1)
    eb = exp_b.reshape(EMB_DIM, 1)

    emb_n = pl.pallas_call(
        _embnorm_kernel,
        grid=(NCB,),
        in_specs=[pl.BlockSpec((CB_TILE, FACT_DIM), lambda cb: (cb, 0))],
        out_specs=pl.BlockSpec((CB_TILE, FACT_DIM), lambda cb: (cb, 0)),
        out_shape=jax.ShapeDtypeStruct((NUM_EMB, FACT_DIM), jnp.float32),
    )(emb_w)

    idx, loss = pl.pallas_call(
        _search_kernel,
        grid=(B,),
        in_specs=[
            pl.BlockSpec((1, EMB_DIM, PIX), lambda b: (b, 0, 0)),
            pl.BlockSpec((NUM_EMB, FACT_DIM), lambda b: (0, 0)),
            pl.BlockSpec((FACT_DIM, EMB_DIM), lambda b: (0, 0)),
            pl.BlockSpec((FACT_DIM, 1), lambda b: (0, 0)),
        ],
        out_specs=[
            pl.BlockSpec((1, 1, PIX), lambda b: (b, 0, 0)),
            pl.BlockSpec((1, 1), lambda b: (0, 0)),
        ],
        out_shape=[
            jax.ShapeDtypeStruct((B, 1, PIX), jnp.int32),
            jax.ShapeDtypeStruct((1, 1), jnp.float32),
        ],
        scratch_shapes=[
            pltpu.VMEM((1, 1), jnp.float32),
        ],
        compiler_params=pltpu.CompilerParams(
            dimension_semantics=("arbitrary",),
        ),
    )(x, emb_n, proj_w, pb)

    lat_raw, counts = _sc_gather_count(idx, emb_n)

    out, perp = pl.pallas_call(
        _expand_kernel,
        grid=(B,),
        in_specs=[
            pl.BlockSpec((1, PIX, FACT_DIM), lambda b: (b, 0, 0)),
            pl.BlockSpec((EMB_DIM, FACT_DIM), lambda b: (0, 0)),
            pl.BlockSpec((EMB_DIM, 1), lambda b: (0, 0)),
            pl.BlockSpec((_NW, NUM_EMB), lambda b: (0, 0)),
        ],
        out_specs=[
            pl.BlockSpec((1, EMB_DIM, PIX), lambda b: (b, 0, 0)),
            pl.BlockSpec((1, 1), lambda b: (0, 0)),
        ],
        out_shape=[
            jax.ShapeDtypeStruct((B, EMB_DIM, PIX), jnp.float32),
            jax.ShapeDtypeStruct((1, 1), jnp.float32),
        ],
        compiler_params=pltpu.CompilerParams(
            dimension_semantics=("arbitrary",),
        ),
    )(lat_raw.reshape(B, PIX, FACT_DIM), exp_w, eb, counts)

    return (out.reshape(B, EMB_DIM, 32, 32), loss[0, 0], perp[0, 0])


# speedup vs baseline: 1.0334x; 1.0043x over previous
import functools

import jax
import jax.numpy as jnp
from jax.experimental import pallas as pl
from jax.experimental.pallas import tpu as pltpu
from jax.experimental.pallas import tpu_sc as plsc

NUM_EMB = 16384
EMB_DIM = 512
FACT_DIM = 32
B = 8
PIX = 1024
NPIX = B * PIX
CB_TILE = 2048
NCB = NUM_EMB // CB_TILE

_PREC = jax.lax.Precision.DEFAULT

_SC_INFO = plsc.get_sparse_core_info()
_NW = _SC_INFO.num_cores * _SC_INFO.num_subcores
_BPW = NPIX // _NW


def _embnorm_kernel(embw_ref, embn_ref):
    ew = embw_ref[...]
    en = jnp.sqrt(jnp.sum(ew * ew, axis=1, keepdims=True))
    embn_ref[...] = ew / jnp.maximum(en, 1e-6)


def _search_kernel(x_ref, embn_ref, projw_ref, projb_ref,
                   idx_ref, loss_ref, loss_scr):
    b = pl.program_id(0)

    @pl.when(b == 0)
    def _init():
        loss_scr[...] = jnp.zeros_like(loss_scr)

    x = x_ref[0]
    enc = jax.lax.dot_general(projw_ref[...], x, (((1,), (0,)), ((), ())),
                              precision=_PREC,
                              preferred_element_type=jnp.float32)
    enc = enc + projb_ref[...]
    n = jnp.sqrt(jnp.sum(enc * enc, axis=0, keepdims=True))
    enc_n = enc / jnp.maximum(n, 1e-6)

    bval = jnp.full((1, PIX), -jnp.inf, jnp.float32)
    bidx = jnp.zeros((1, PIX), jnp.int32)
    for j in range(NCB):
        emb_n = embn_ref[j * CB_TILE:(j + 1) * CB_TILE, :]
        cos = jax.lax.dot_general(emb_n, enc_n, (((1,), (0,)), ((), ())),
                                  precision=_PREC,
                                  preferred_element_type=jnp.float32)
        lmax = jnp.max(cos, axis=0, keepdims=True)
        lidx = jnp.argmax(cos, axis=0).reshape(1, PIX) + j * CB_TILE
        better = lmax > bval
        bidx = jnp.where(better, lidx, bidx)
        bval = jnp.where(better, lmax, bval)

    idx_ref[0] = bidx
    loss_scr[...] += jnp.sum(2.0 - 2.0 * bval).reshape(1, 1)

    @pl.when(b == B - 1)
    def _finish_all():
        loss_ref[...] = loss_scr[...] / (B * FACT_DIM * PIX)


def _sc_body(idx_hbm, embw_hbm, lat_hbm, cnt_hbm,
             idx_v, rows_v, cnt_v, sem):
    wid = jax.lax.axis_index("s") * _SC_INFO.num_cores + jax.lax.axis_index("c")
    per_b = PIX // _BPW
    b = wid // per_b
    off = (wid % per_b) * _BPW
    pltpu.sync_copy(idx_hbm.at[b, 0, pl.ds(off, _BPW)], idx_v)
    pltpu.async_copy(embw_hbm.at[idx_v], rows_v, sem).wait()
    pltpu.sync_copy(rows_v, lat_hbm.at[pl.ds(wid * _BPW, _BPW)])
    zeros16 = jnp.zeros((16,), jnp.float32)

    def zbody(i, carry):
        cnt_v[pl.ds(i * 16, 16)] = zeros16
        return carry

    jax.lax.fori_loop(0, NUM_EMB // 16, zbody, 0)
    ones = jnp.ones((16,), jnp.float32)

    def body(i, carry):
        v = idx_v[pl.ds(i * 16, 16)]
        plsc.addupdate_scatter(cnt_v, [v], ones)
        return carry

    jax.lax.fori_loop(0, _BPW // 16, body, 0)
    pltpu.sync_copy(cnt_v, cnt_hbm.at[wid])


@functools.partial(
    pl.kernel,
    mesh=plsc.VectorSubcoreMesh(core_axis_name="c", subcore_axis_name="s"),
    out_type=[
        jax.ShapeDtypeStruct((NPIX, FACT_DIM), jnp.float32),
        jax.ShapeDtypeStruct((_NW, NUM_EMB), jnp.float32),
    ],
    scratch_types=[
        pltpu.VMEM((_BPW,), jnp.int32),
        pltpu.VMEM((_BPW, FACT_DIM), jnp.float32),
        pltpu.VMEM((NUM_EMB,), jnp.float32),
        pltpu.SemaphoreType.DMA,
    ],
    compiler_params=pltpu.CompilerParams(needs_layout_passes=False,
                                         use_tc_tiling_on_sc=False),
)
def _sc_gather_count(idx_hbm, embw_hbm, lat_hbm, cnt_hbm,
                     idx_v, rows_v, cnt_v, sem):
    _sc_body(idx_hbm, embw_hbm, lat_hbm, cnt_hbm,
             idx_v, rows_v, cnt_v, sem)


def _expand_kernel(lat_ref, expw_ref, expb_ref, cnt_ref,
                   out_ref, perp_ref):
    b = pl.program_id(0)
    lat_n = lat_ref[0]
    o = jax.lax.dot_general(expw_ref[...], lat_n, (((1,), (1,)), ((), ())),
                            precision=_PREC,
                            preferred_element_type=jnp.float32)
    out_ref[0] = o + expb_ref[...]

    @pl.when(b == 0)
    def _entropy():
        tot = jnp.sum(cnt_ref[...], axis=0, keepdims=True)
        u = tot / float(NPIX)
        ent = jnp.sum(-u * jnp.log(u + 1e-6))
        perp_ref[...] = jnp.exp(ent).reshape(1, 1)


def kernel(encodings, emb_w, proj_w, proj_b, exp_w, exp_b):
    x = encodings.reshape(B, EMB_DIM, PIX)
    pb = proj_b.reshape(FACT_DIM, 1)
    eb = exp_b.reshape(EMB_DIM, 1)

    emb_n = pl.pallas_call(
        _embnorm_kernel,
        grid=(NCB,),
        in_specs=[pl.BlockSpec((CB_TILE, FACT_DIM), lambda cb: (cb, 0))],
        out_specs=pl.BlockSpec((CB_TILE, FACT_DIM), lambda cb: (cb, 0)),
        out_shape=jax.ShapeDtypeStruct((NUM_EMB, FACT_DIM), jnp.float32),
    )(emb_w)

    idx, loss = pl.pallas_call(
        _search_kernel,
        grid=(B,),
        in_specs=[
            pl.BlockSpec((1, EMB_DIM, PIX), lambda b: (b, 0, 0)),
            pl.BlockSpec((NUM_EMB, FACT_DIM), lambda b: (0, 0)),
            pl.BlockSpec((FACT_DIM, EMB_DIM), lambda b: (0, 0)),
            pl.BlockSpec((FACT_DIM, 1), lambda b: (0, 0)),
        ],
        out_specs=[
            pl.BlockSpec((1, 1, PIX), lambda b: (b, 0, 0)),
            pl.BlockSpec((1, 1), lambda b: (0, 0)),
        ],
        out_shape=[
            jax.ShapeDtypeStruct((B, 1, PIX), jnp.int32),
            jax.ShapeDtypeStruct((1, 1), jnp.float32),
        ],
        scratch_shapes=[
            pltpu.VMEM((1, 1), jnp.float32),
        ],
        compiler_params=pltpu.CompilerParams(
            dimension_semantics=("arbitrary",),
        ),
    )(x, emb_n, proj_w, pb)

    lat_raw, counts = _sc_gather_count(idx, emb_n)

    out, perp = pl.pallas_call(
        _expand_kernel,
        grid=(B,),
        in_specs=[
            pl.BlockSpec((1, PIX, FACT_DIM), lambda b: (b, 0, 0)),
            pl.BlockSpec((EMB_DIM, FACT_DIM), lambda b: (0, 0)),
            pl.BlockSpec((EMB_DIM, 1), lambda b: (0, 0)),
            pl.BlockSpec((_NW, NUM_EMB), lambda b: (0, 0)),
        ],
        out_specs=[
            pl.BlockSpec((1, EMB_DIM, PIX), lambda b: (b, 0, 0)),
            pl.BlockSpec((1, 1), lambda b: (0, 0)),
        ],
        out_shape=[
            jax.ShapeDtypeStruct((B, EMB_DIM, PIX), jnp.float32),
            jax.ShapeDtypeStruct((1, 1), jnp.float32),
        ],
        compiler_params=pltpu.CompilerParams(
            dimension_semantics=("arbitrary",),
        ),
    )(lat_raw.reshape(B, PIX, FACT_DIM), exp_w, eb, counts)

    return (out.reshape(B, EMB_DIM, 32, 32), loss[0, 0], perp[0, 0])


# speedup vs baseline: 1.0938x; 1.0585x over previous
import functools

import jax
import jax.numpy as jnp
from jax.experimental import pallas as pl
from jax.experimental.pallas import tpu as pltpu
from jax.experimental.pallas import tpu_sc as plsc

NUM_EMB = 16384
EMB_DIM = 512
FACT_DIM = 32
B = 8
PIX = 1024
NPIX = B * PIX
CB_TILE = 2048
NCB = NUM_EMB // CB_TILE

_PREC = jax.lax.Precision.DEFAULT

_SC_INFO = plsc.get_sparse_core_info()
_NW = _SC_INFO.num_cores * _SC_INFO.num_subcores
_BPW = NPIX // _NW


def _search_kernel(x_ref, embw_ref, projw_ref, projb_ref,
                   idx_ref, loss_ref, embn_scr, loss_scr):
    b = pl.program_id(0)

    @pl.when(b == 0)
    def _init():
        loss_scr[...] = jnp.zeros_like(loss_scr)
        ew = embw_ref[...]
        en = jnp.sqrt(jnp.sum(ew * ew, axis=1, keepdims=True))
        embn_scr[...] = ew / jnp.maximum(en, 1e-6)

    x = x_ref[0]
    enc = jax.lax.dot_general(projw_ref[...], x, (((1,), (0,)), ((), ())),
                              precision=_PREC,
                              preferred_element_type=jnp.float32)
    enc = enc + projb_ref[...]
    n = jnp.sqrt(jnp.sum(enc * enc, axis=0, keepdims=True))
    enc_n = enc / jnp.maximum(n, 1e-6)

    bval = jnp.full((1, PIX), -jnp.inf, jnp.float32)
    bidx = jnp.zeros((1, PIX), jnp.int32)
    for j in range(NCB):
        emb_n = embn_scr[j * CB_TILE:(j + 1) * CB_TILE, :]
        cos = jax.lax.dot_general(emb_n, enc_n, (((1,), (0,)), ((), ())),
                                  precision=_PREC,
                                  preferred_element_type=jnp.float32)
        lmax = jnp.max(cos, axis=0, keepdims=True)
        lidx = jnp.argmax(cos, axis=0).reshape(1, PIX) + j * CB_TILE
        better = lmax > bval
        bidx = jnp.where(better, lidx, bidx)
        bval = jnp.where(better, lmax, bval)

    idx_ref[0] = bidx
    loss_scr[...] += jnp.sum(2.0 - 2.0 * bval).reshape(1, 1)

    @pl.when(b == B - 1)
    def _finish_all():
        loss_ref[...] = loss_scr[...] / (B * FACT_DIM * PIX)


def _sc_body(idx_hbm, embw_hbm, lat_hbm, cnt_hbm,
             idx_v, rows_v, cnt_v, sem):
    wid = jax.lax.axis_index("s") * _SC_INFO.num_cores + jax.lax.axis_index("c")
    per_b = PIX // _BPW
    b = wid // per_b
    off = (wid % per_b) * _BPW
    pltpu.sync_copy(idx_hbm.at[b, 0, pl.ds(off, _BPW)], idx_v)
    pltpu.async_copy(embw_hbm.at[idx_v], rows_v, sem).wait()
    pltpu.sync_copy(rows_v, lat_hbm.at[pl.ds(wid * _BPW, _BPW)])
    zeros16 = jnp.zeros((16,), jnp.float32)

    def zbody(i, carry):
        for k in range(16):
            cnt_v[pl.ds(i * 256 + k * 16, 16)] = zeros16
        return carry

    jax.lax.fori_loop(0, NUM_EMB // 256, zbody, 0)
    ones = jnp.ones((16,), jnp.float32)

    def body(i, carry):
        v = idx_v[pl.ds(i * 16, 16)]
        plsc.addupdate_scatter(cnt_v, [v], ones)
        return carry

    jax.lax.fori_loop(0, _BPW // 16, body, 0)
    pltpu.sync_copy(cnt_v, cnt_hbm.at[wid])


@functools.partial(
    pl.kernel,
    mesh=plsc.VectorSubcoreMesh(core_axis_name="c", subcore_axis_name="s"),
    out_type=[
        jax.ShapeDtypeStruct((NPIX, FACT_DIM), jnp.float32),
        jax.ShapeDtypeStruct((_NW, NUM_EMB), jnp.float32),
    ],
    scratch_types=[
        pltpu.VMEM((_BPW,), jnp.int32),
        pltpu.VMEM((_BPW, FACT_DIM), jnp.float32),
        pltpu.VMEM((NUM_EMB,), jnp.float32),
        pltpu.SemaphoreType.DMA,
    ],
    compiler_params=pltpu.CompilerParams(needs_layout_passes=False,
                                         use_tc_tiling_on_sc=False),
)
def _sc_gather_count(idx_hbm, embw_hbm, lat_hbm, cnt_hbm,
                     idx_v, rows_v, cnt_v, sem):
    _sc_body(idx_hbm, embw_hbm, lat_hbm, cnt_hbm,
             idx_v, rows_v, cnt_v, sem)


def _expand_kernel(lat_ref, expw_ref, expb_ref, cnt_ref,
                   out_ref, perp_ref):
    b = pl.program_id(0)
    lraw = lat_ref[0]
    ln = jnp.sqrt(jnp.sum(lraw * lraw, axis=1, keepdims=True))
    lat_n = lraw / jnp.maximum(ln, 1e-6)
    o = jax.lax.dot_general(expw_ref[...], lat_n, (((1,), (1,)), ((), ())),
                            precision=_PREC,
                            preferred_element_type=jnp.float32)
    out_ref[0] = o + expb_ref[...]

    @pl.when(b == 0)
    def _entropy():
        tot = jnp.sum(cnt_ref[...], axis=0, keepdims=True)
        u = tot / float(NPIX)
        ent = jnp.sum(-u * jnp.log(u + 1e-6))
        perp_ref[...] = jnp.exp(ent).reshape(1, 1)


def kernel(encodings, emb_w, proj_w, proj_b, exp_w, exp_b):
    x = encodings.reshape(B, EMB_DIM, PIX)
    pb = proj_b.reshape(FACT_DIM, 1)
    eb = exp_b.reshape(EMB_DIM, 1)

    idx, loss = pl.pallas_call(
        _search_kernel,
        grid=(B,),
        in_specs=[
            pl.BlockSpec((1, EMB_DIM, PIX), lambda b: (b, 0, 0)),
            pl.BlockSpec((NUM_EMB, FACT_DIM), lambda b: (0, 0)),
            pl.BlockSpec((FACT_DIM, EMB_DIM), lambda b: (0, 0)),
            pl.BlockSpec((FACT_DIM, 1), lambda b: (0, 0)),
        ],
        out_specs=[
            pl.BlockSpec((1, 1, PIX), lambda b: (b, 0, 0)),
            pl.BlockSpec((1, 1), lambda b: (0, 0)),
        ],
        out_shape=[
            jax.ShapeDtypeStruct((B, 1, PIX), jnp.int32),
            jax.ShapeDtypeStruct((1, 1), jnp.float32),
        ],
        scratch_shapes=[
            pltpu.VMEM((NUM_EMB, FACT_DIM), jnp.float32),
            pltpu.VMEM((1, 1), jnp.float32),
        ],
        compiler_params=pltpu.CompilerParams(
            dimension_semantics=("arbitrary",),
        ),
    )(x, emb_w, proj_w, pb)

    lat_raw, counts = _sc_gather_count(idx, emb_w)

    out, perp = pl.pallas_call(
        _expand_kernel,
        grid=(B,),
        in_specs=[
            pl.BlockSpec((1, PIX, FACT_DIM), lambda b: (b, 0, 0)),
            pl.BlockSpec((EMB_DIM, FACT_DIM), lambda b: (0, 0)),
            pl.BlockSpec((EMB_DIM, 1), lambda b: (0, 0)),
            pl.BlockSpec((_NW, NUM_EMB), lambda b: (0, 0)),
        ],
        out_specs=[
            pl.BlockSpec((1, EMB_DIM, PIX), lambda b: (b, 0, 0)),
            pl.BlockSpec((1, 1), lambda b: (0, 0)),
        ],
        out_shape=[
            jax.ShapeDtypeStruct((B, EMB_DIM, PIX), jnp.float32),
            jax.ShapeDtypeStruct((1, 1), jnp.float32),
        ],
        compiler_params=pltpu.CompilerParams(
            dimension_semantics=("arbitrary",),
        ),
    )(lat_raw.reshape(B, PIX, FACT_DIM), exp_w, eb, counts)

    return (out.reshape(B, EMB_DIM, 32, 32), loss[0, 0], perp[0, 0])
